# Initial kernel scaffold; baseline (speedup 1.0000x reference)
import jax
import jax.numpy as jnp
from jax.experimental import pallas as pl


def kernel(x, W):
    raise NotImplementedError("write your pallas kernel here")


# speedup vs baseline: 1.1273x; 1.1273x over previous
import functools

import jax
import jax.numpy as jnp
from jax import lax
from jax.experimental import pallas as pl
from jax.experimental.pallas import tpu as pltpu

_HIDDEN = 4096
_NUM_EXPERTS = 64
_TOP_K = 8
_TOKENS = 8192
_BLOCK_T = 512


def _router_body(x_ref, w_ref, scores_ref, weights_ref, indices_ref):
    x = x_ref[...]
    w = w_ref[...]
    logits = lax.dot_general(
        x, w,
        dimension_numbers=(((1,), (1,)), ((), ())),
        preferred_element_type=jnp.float32,
    )
    m = jnp.max(logits, axis=-1, keepdims=True)
    e = jnp.exp(logits - m)
    s = e / jnp.sum(e, axis=-1, keepdims=True)
    scores_ref[...] = s

    col = lax.broadcasted_iota(jnp.int32, s.shape, 1)
    remaining = s
    w_cols = []
    i_cols = []
    for _ in range(_TOP_K):
        cur = jnp.max(remaining, axis=-1, keepdims=True)
        hit = remaining == cur
        idx = jnp.min(jnp.where(hit, col, _NUM_EXPERTS), axis=-1, keepdims=True)
        w_cols.append(cur)
        i_cols.append(idx)
        remaining = jnp.where(col == idx, -jnp.inf, remaining)
    weights_ref[...] = jnp.concatenate(w_cols, axis=1)
    indices_ref[...] = jnp.concatenate(i_cols, axis=1)


@jax.jit
def kernel(x, W):
    tokens = x.shape[0]
    grid = tokens // _BLOCK_T
    scores, weights, indices = pl.pallas_call(
        _router_body,
        grid=(grid,),
        in_specs=[
            pl.BlockSpec((_BLOCK_T, _HIDDEN), lambda i: (i, 0)),
            pl.BlockSpec((_NUM_EXPERTS, _HIDDEN), lambda i: (0, 0)),
        ],
        out_specs=[
            pl.BlockSpec((_BLOCK_T, _NUM_EXPERTS), lambda i: (i, 0)),
            pl.BlockSpec((_BLOCK_T, _TOP_K), lambda i: (i, 0)),
            pl.BlockSpec((_BLOCK_T, _TOP_K), lambda i: (i, 0)),
        ],
        out_shape=[
            jax.ShapeDtypeStruct((tokens, _NUM_EXPERTS), jnp.float32),
            jax.ShapeDtypeStruct((tokens, _TOP_K), jnp.float32),
            jax.ShapeDtypeStruct((tokens, _TOP_K), jnp.int32),
        ],
    )(x, W)
    return (scores, weights, indices)
